# baseline (speedup 1.0000x reference)
import functools

import jax
import jax.numpy as jnp
from jax.experimental import pallas as pl
from jax.experimental.pallas import tpu as pltpu

_MAX_DISP = 4
_NS = 2 * _MAX_DISP + 1


def _corr_kernel(ref_ref, q_ref, out_ref, qy_ref, *, n_chunks, inv_c):
    k = pl.program_id(1)
    _, cc, h, w = ref_ref.shape

    @pl.when(k == 0)
    def _():
        out_ref[...] = jnp.zeros_like(out_ref)

    wt = min(128, w)
    n_xt = w // wt
    for dy in range(_NS):
        slot = 0
        qy_ref[slot] = q_ref[0, :, dy:dy + h, :]
        for dx in range(_NS):
            i = dy * _NS + dx
            for xt in range(n_xt):
                xs = slice(xt * wt, (xt + 1) * wt)
                xq = slice(xt * wt + dx, (xt + 1) * wt + dx)
                acc = ref_ref[0, 0, :, xs] * qy_ref[slot, 0, :, xq]
                for ci in range(1, cc):
                    acc = acc + (ref_ref[0, ci, :, xs]
                                 * qy_ref[slot, ci, :, xq])
                out_ref[0, i, :, xs] += acc

    @pl.when(k == n_chunks - 1)
    def _():
        out_ref[...] = out_ref[...] * inv_c


def kernel(reference_features, query_features):
    b, c, h, w = reference_features.shape
    p = _MAX_DISP
    q = jnp.pad(query_features, ((0, 0), (0, 0), (p, p), (p, p)))

    cc = min(32, c)
    n_chunks = c // cc
    d = _NS * _NS

    return pl.pallas_call(
        functools.partial(_corr_kernel, n_chunks=n_chunks, inv_c=1.0 / c),
        grid=(b, n_chunks),
        in_specs=[
            pl.BlockSpec((1, cc, h, w), lambda bi, ki: (bi, ki, 0, 0)),
            pl.BlockSpec((1, cc, h + 2 * p, w + 2 * p),
                         lambda bi, ki: (bi, ki, 0, 0)),
        ],
        out_specs=pl.BlockSpec((1, d, h, w), lambda bi, ki: (bi, 0, 0, 0)),
        out_shape=jax.ShapeDtypeStruct((b, d, h, w), jnp.float32),
        scratch_shapes=[
            pltpu.VMEM((1, cc, h, w + 2 * p), jnp.float32),
        ],
        compiler_params=pltpu.CompilerParams(
            dimension_semantics=("parallel", "arbitrary"),
            vmem_limit_bytes=56 * 1024 * 1024,
        ),
        name="corr_cost_volume",
    )(reference_features, q)


# speedup vs baseline: 2.7629x; 2.7629x over previous
import functools

import jax
import jax.numpy as jnp
from jax.experimental import pallas as pl
from jax.experimental.pallas import tpu as pltpu

_MAX_DISP = 4
_NS = 2 * _MAX_DISP + 1


def _corr_kernel(ref_ref, q_ref, out_ref, qx_ref, *, n_chunks, inv_c):
    k = pl.program_id(1)
    _, cc, h, w = ref_ref.shape

    @pl.when(k == 0)
    def _():
        out_ref[...] = jnp.zeros_like(out_ref)

    wt = min(128, w)
    n_xt = w // wt
    for dx in range(_NS):
        slot = dx % 2
        qx_ref[slot] = q_ref[0, :, :, dx:dx + w]
        for dy in range(_NS):
            i = dy * _NS + dx
            for xt in range(n_xt):
                xs = slice(xt * wt, (xt + 1) * wt)
                half = cc // 2
                acc0 = (ref_ref[0, 0, :, xs]
                        * qx_ref[slot, 0, dy:dy + h, xs])
                acc1 = (ref_ref[0, half, :, xs]
                        * qx_ref[slot, half, dy:dy + h, xs])
                for ci in range(1, half):
                    acc0 = acc0 + (ref_ref[0, ci, :, xs]
                                   * qx_ref[slot, ci, dy:dy + h, xs])
                    acc1 = acc1 + (ref_ref[0, half + ci, :, xs]
                                   * qx_ref[slot, half + ci, dy:dy + h, xs])
                out_ref[0, i, :, xs] += acc0 + acc1

    @pl.when(k == n_chunks - 1)
    def _():
        out_ref[...] = out_ref[...] * inv_c


def kernel(reference_features, query_features):
    b, c, h, w = reference_features.shape
    p = _MAX_DISP
    q = jnp.pad(query_features, ((0, 0), (0, 0), (p, p), (p, p)))

    cc = min(32, c)
    n_chunks = c // cc
    d = _NS * _NS

    return pl.pallas_call(
        functools.partial(_corr_kernel, n_chunks=n_chunks, inv_c=1.0 / c),
        grid=(b, n_chunks),
        in_specs=[
            pl.BlockSpec((1, cc, h, w), lambda bi, ki: (bi, ki, 0, 0)),
            pl.BlockSpec((1, cc, h + 2 * p, w + 2 * p),
                         lambda bi, ki: (bi, ki, 0, 0)),
        ],
        out_specs=pl.BlockSpec((1, d, h, w), lambda bi, ki: (bi, 0, 0, 0)),
        out_shape=jax.ShapeDtypeStruct((b, d, h, w), jnp.float32),
        scratch_shapes=[
            pltpu.VMEM((2, cc, h + 2 * p, w), jnp.float32),
        ],
        compiler_params=pltpu.CompilerParams(
            dimension_semantics=("parallel", "arbitrary"),
            vmem_limit_bytes=56 * 1024 * 1024,
        ),
        name="corr_cost_volume",
    )(reference_features, q)
